# Initial kernel scaffold; baseline (speedup 1.0000x reference)
import jax
import jax.numpy as jnp
from jax.experimental import pallas as pl


def kernel(x, W1, W2):
    raise NotImplementedError("write your pallas kernel here")


# speedup vs baseline: 7.1262x; 7.1262x over previous
import functools

import jax
import jax.numpy as jnp
from jax.experimental import pallas as pl
from jax.experimental.pallas import tpu as pltpu

N_TOK = 32768
D_IN = 768
N_EXP = 64
K_SEL = 8
BLK = 1024
GRID = N_TOK // BLK


def _router_body(x_ref, w1_ref, w2_ref,
                 idx_ref, scr_ref, loss_ref, load_ref, imp_ref,
                 load_acc):
    i = pl.program_id(0)

    h = jnp.tanh(jax.lax.dot_general(
        x_ref[...], w1_ref[...], (((1,), (1,)), ((), ())),
        preferred_element_type=jnp.float32))
    logits = jax.lax.dot_general(
        h, w2_ref[...], (((1,), (1,)), ((), ())),
        preferred_element_type=jnp.float32)

    iota = jax.lax.broadcasted_iota(jnp.int32, logits.shape, 1)
    l = logits
    vals, idxs = [], []
    P = jnp.zeros_like(logits)
    v0 = None
    Z = None
    for r in range(K_SEL):
        m = jnp.max(l, axis=1, keepdims=True)
        idx = jnp.min(jnp.where(l == m, iota, N_EXP), axis=1,
                      keepdims=True)
        onehot = iota == idx
        if r == 0:
            v0 = m
        e = jnp.exp(m - v0)
        P = P + jnp.where(onehot, e, 0.0)
        Z = e if Z is None else Z + e
        vals.append(m)
        idxs.append(idx)
        l = jnp.where(onehot, -jnp.inf, l)

    idx_ref[...] = jnp.concatenate(idxs, axis=1)
    scr_ref[...] = jnp.concatenate(
        [jnp.exp(v - v0) for v in vals], axis=1) / Z

    scores_f = P / Z
    imp_part = jnp.sum(scores_f, axis=0, keepdims=True)
    load_part = jnp.sum((scores_f > 0).astype(jnp.float32), axis=0,
                        keepdims=True)

    @pl.when(i == 0)
    def _init():
        imp_ref[...] = jnp.zeros_like(imp_ref)
        load_acc[...] = jnp.zeros_like(load_acc)

    imp_ref[...] += imp_part
    load_acc[...] += load_part

    @pl.when(i == GRID - 1)
    def _finish():
        def cv2(v):
            mean = jnp.mean(v)
            var = jnp.sum((v - mean) ** 2) / (v.size - 1)
            return var / (mean * mean + 1e-10)

        imp = imp_ref[...]
        load_f = load_acc[...]
        load_ref[...] = load_f.astype(jnp.int32)
        loss_ref[...] = jnp.full((1, 1), 0.01) * (cv2(imp) + cv2(load_f))


@jax.jit
def kernel(x, W1, W2):
    out_shapes = (
        jax.ShapeDtypeStruct((N_TOK, K_SEL), jnp.int32),
        jax.ShapeDtypeStruct((N_TOK, K_SEL), jnp.float32),
        jax.ShapeDtypeStruct((1, 1), jnp.float32),
        jax.ShapeDtypeStruct((1, N_EXP), jnp.int32),
        jax.ShapeDtypeStruct((1, N_EXP), jnp.float32),
    )
    grid = (GRID,)
    in_specs = [
        pl.BlockSpec((BLK, D_IN), lambda i: (i, 0)),
        pl.BlockSpec((N_EXP, D_IN), lambda i: (0, 0)),
        pl.BlockSpec((N_EXP, N_EXP), lambda i: (0, 0)),
    ]
    out_specs = (
        pl.BlockSpec((BLK, K_SEL), lambda i: (i, 0)),
        pl.BlockSpec((BLK, K_SEL), lambda i: (i, 0)),
        pl.BlockSpec((1, 1), lambda i: (0, 0)),
        pl.BlockSpec((1, N_EXP), lambda i: (0, 0)),
        pl.BlockSpec((1, N_EXP), lambda i: (0, 0)),
    )
    idx, scr, loss, load, imp = pl.pallas_call(
        _router_body,
        grid=grid,
        in_specs=in_specs,
        out_specs=out_specs,
        out_shape=out_shapes,
        scratch_shapes=[pltpu.VMEM((1, N_EXP), jnp.float32)],
    )(x, W1, W2)
    return (idx, scr, loss.reshape(()), load.reshape(N_EXP),
            imp.reshape(N_EXP))


# speedup vs baseline: 7.6360x; 1.0715x over previous
import functools

import jax
import jax.numpy as jnp
from jax import lax
from jax.experimental import pallas as pl
from jax.experimental.pallas import tpu as pltpu
from jax.experimental.pallas import tpu_sc as plsc

N_TOK = 32768
D_IN = 768
N_EXP = 64
K_SEL = 8

NW = 32
ROWS_PER_W = N_TOK // NW
CHUNK = 128
NCHUNK = ROWS_PER_W // CHUNK
NGRP = CHUNK // 16
UNROLL = 8

MM_BLK = 2048


def _gate_body(x_ref, w1_ref, w2_ref, out_ref):
    h = jnp.tanh(lax.dot_general(
        x_ref[...], w1_ref[...], (((1,), (1,)), ((), ())),
        preferred_element_type=jnp.float32))
    out_ref[...] = lax.dot_general(
        w2_ref[...], h, (((1,), (1,)), ((), ())),
        preferred_element_type=jnp.float32)


def _gate(x, W1, W2):
    return pl.pallas_call(
        _gate_body,
        grid=(N_TOK // MM_BLK,),
        in_specs=[
            pl.BlockSpec((MM_BLK, D_IN), lambda i: (i, 0)),
            pl.BlockSpec((N_EXP, D_IN), lambda i: (0, 0)),
            pl.BlockSpec((N_EXP, N_EXP), lambda i: (0, 0)),
        ],
        out_specs=pl.BlockSpec((N_EXP, MM_BLK), lambda i: (0, i)),
        out_shape=jax.ShapeDtypeStruct((N_EXP, N_TOK), jnp.float32),
    )(x, W1, W2)


def _merge(va, ia, vb, ib):
    m = vb > va
    return jnp.where(m, vb, va), jnp.where(m, ib, ia)


def _route_body(lt_hbm, idx_hbm, scr_hbm, imp_hbm, load_hbm,
                buf0, buf1, wbuf, idxf, scrf, impf, loadf, dsem):
    c = lax.axis_index("c")
    s = lax.axis_index("s")
    wid = s * 2 + c
    base = wid * ROWS_PER_W
    lanes = lax.iota(jnp.int32, 16)
    zeros16 = jnp.zeros((16,), jnp.float32)
    neg_inf = jnp.full((16,), -jnp.inf, jnp.float32)

    def zbody(i, carry):
        impf[pl.ds(i * 16, 16)] = zeros16
        loadf[pl.ds(i * 16, 16)] = zeros16
        return carry

    lax.fori_loop(0, N_EXP, zbody, 0)

    bufs = (buf0, buf1)

    def start_in(ci, buf):
        return pltpu.async_copy(
            lt_hbm.at[:, pl.ds(base + ci * CHUNK, CHUNK)], buf, dsem)

    def process_chunk(ci, buf):
        def gbody(g, carry):
            col0 = g * 16

            def fbody(jj, fcarry):
                for u in range(UNROLL):
                    j = jj * UNROLL + u
                    wbuf[pl.ds(j * 16, 16)] = buf[j, pl.ds(col0, 16)]
                return fcarry

            lax.fori_loop(0, N_EXP // UNROLL, fbody, 0)

            bests, bidxs = [], []
            for _r in range(K_SEL):
                def jbody(jj, st):
                    best, bidx = st
                    vals = []
                    for u in range(UNROLL):
                        j = jj * UNROLL + u
                        vals.append((wbuf[pl.ds(j * 16, 16)],
                                     jnp.full((16,), j, jnp.int32)))
                    while len(vals) > 1:
                        nxt = []
                        for p in range(0, len(vals), 2):
                            nxt.append(_merge(*vals[p], *vals[p + 1]))
                        vals = nxt
                    return _merge(best, bidx, *vals[0])

                best, bidx = lax.fori_loop(
                    0, N_EXP // UNROLL, jbody,
                    (neg_inf, jnp.zeros((16,), jnp.int32)))
                plsc.store_scatter(wbuf, [bidx * 16 + lanes], neg_inf)
                bests.append(best)
                bidxs.append(bidx)

            v0 = bests[0]
            es = [jnp.exp(b - v0) for b in bests]
            z = es[0]
            for e in es[1:]:
                z = z + e
            rowl = col0 + lanes
            for r in range(K_SEL):
                score = es[r] / z
                addr = rowl * K_SEL + r
                plsc.store_scatter(idxf, [addr], bidxs[r])
                plsc.store_scatter(scrf, [addr], score)
                plsc.addupdate_scatter(impf, [lanes * N_EXP + bidxs[r]], score)
                plsc.addupdate_scatter(
                    loadf, [lanes * N_EXP + bidxs[r]],
                    jnp.where(score > 0, jnp.float32(1), jnp.float32(0)))
            return carry

        lax.fori_loop(0, NGRP, gbody, 0)
        row0 = base + ci * CHUNK
        pltpu.sync_copy(idxf, idx_hbm.at[pl.ds(row0 * K_SEL, CHUNK * K_SEL)])
        pltpu.sync_copy(scrf, scr_hbm.at[pl.ds(row0 * K_SEL, CHUNK * K_SEL)])

    pending = start_in(0, bufs[0])
    for ci in range(NCHUNK):
        pending.wait()
        if ci + 1 < NCHUNK:
            pending = start_in(ci + 1, bufs[(ci + 1) % 2])
        process_chunk(ci, bufs[ci % 2])

    pltpu.sync_copy(impf, imp_hbm.at[pl.ds(wid * 16 * N_EXP, 16 * N_EXP)])
    pltpu.sync_copy(loadf, load_hbm.at[pl.ds(wid * 16 * N_EXP, 16 * N_EXP)])


def _route(lt):
    f = pl.kernel(
        _route_body,
        out_type=(
            jax.ShapeDtypeStruct((N_TOK * K_SEL,), jnp.int32),
            jax.ShapeDtypeStruct((N_TOK * K_SEL,), jnp.float32),
            jax.ShapeDtypeStruct((NW * 16 * N_EXP,), jnp.float32),
            jax.ShapeDtypeStruct((NW * 16 * N_EXP,), jnp.float32),
        ),
        mesh=plsc.VectorSubcoreMesh(core_axis_name="c", subcore_axis_name="s"),
        compiler_params=pltpu.CompilerParams(needs_layout_passes=False),
        scratch_types=[
            pltpu.VMEM((N_EXP, CHUNK), jnp.float32),
            pltpu.VMEM((N_EXP, CHUNK), jnp.float32),
            pltpu.VMEM((16 * N_EXP,), jnp.float32),
            pltpu.VMEM((CHUNK * K_SEL,), jnp.int32),
            pltpu.VMEM((CHUNK * K_SEL,), jnp.float32),
            pltpu.VMEM((16 * N_EXP,), jnp.float32),
            pltpu.VMEM((16 * N_EXP,), jnp.float32),
            pltpu.SemaphoreType.DMA,
        ],
    )
    return f(lt)


def _combine_body(imp_ref, load_ref, loss_ref, load_out, imp_out):
    imp = jnp.sum(imp_ref[...], axis=0, keepdims=True)
    loadf = jnp.sum(load_ref[...], axis=0, keepdims=True)
    imp_out[...] = imp
    load_out[...] = loadf.astype(jnp.int32)

    def cv2(v):
        mean = jnp.mean(v)
        var = jnp.sum((v - mean) ** 2) / (v.size - 1)
        return var / (mean * mean + 1e-10)

    loss_ref[...] = jnp.full((1, 1), 0.01) * (cv2(imp) + cv2(loadf))


def _combine(imp_part, load_part):
    return pl.pallas_call(
        _combine_body,
        out_shape=(
            jax.ShapeDtypeStruct((1, 1), jnp.float32),
            jax.ShapeDtypeStruct((1, N_EXP), jnp.int32),
            jax.ShapeDtypeStruct((1, N_EXP), jnp.float32),
        ),
    )(imp_part, load_part)


@jax.jit
def kernel(x, W1, W2):
    lt = _gate(x, W1, W2)
    idxw, scrw, imp_part, load_part = _route(lt)
    loss, load, imp = _combine(imp_part.reshape(NW * 16, N_EXP),
                               load_part.reshape(NW * 16, N_EXP))
    return (idxw.reshape(N_TOK, K_SEL), scrw.reshape(N_TOK, K_SEL),
            loss.reshape(()), load.reshape(N_EXP), imp.reshape(N_EXP))


# speedup vs baseline: 9.1358x; 1.1964x over previous
import functools

import jax
import jax.numpy as jnp
from jax import lax
from jax.experimental import pallas as pl
from jax.experimental.pallas import tpu as pltpu
from jax.experimental.pallas import tpu_sc as plsc

N_TOK = 32768
D_IN = 768
N_EXP = 64
K_SEL = 8

NW = 32
ROWS_PER_W = N_TOK // NW
CHUNK = 128
NCHUNK = ROWS_PER_W // CHUNK
NGRP = CHUNK // 16
UNROLL = 8

MM_BLK = 2048


def _gate_body(x_ref, w1_ref, w2_ref, out_ref):
    h = jnp.tanh(lax.dot_general(
        x_ref[...], w1_ref[...], (((1,), (1,)), ((), ())),
        preferred_element_type=jnp.float32))
    out_ref[...] = lax.dot_general(
        w2_ref[...], h, (((1,), (1,)), ((), ())),
        preferred_element_type=jnp.float32)


def _gate(x, W1, W2):
    return pl.pallas_call(
        _gate_body,
        grid=(N_TOK // MM_BLK,),
        in_specs=[
            pl.BlockSpec((MM_BLK, D_IN), lambda i: (i, 0)),
            pl.BlockSpec((N_EXP, D_IN), lambda i: (0, 0)),
            pl.BlockSpec((N_EXP, N_EXP), lambda i: (0, 0)),
        ],
        out_specs=pl.BlockSpec((N_EXP, MM_BLK), lambda i: (0, i)),
        out_shape=jax.ShapeDtypeStruct((N_EXP, N_TOK), jnp.float32),
    )(x, W1, W2)


def _merge(va, ia, vb, ib):
    m = vb > va
    return jnp.where(m, vb, va), jnp.where(m, ib, ia)


def _route_body(lt_hbm, idx_hbm, scr_hbm, imp_hbm, load_hbm,
                buf0, buf1, idxb, scrb, imp2d, load2d, dsem):
    c = lax.axis_index("c")
    s = lax.axis_index("s")
    wid = s * 2 + c
    base = wid * ROWS_PER_W
    lanes = lax.iota(jnp.int32, 16)
    zeros16 = jnp.zeros((16,), jnp.float32)
    neg_inf = jnp.full((16,), -jnp.inf, jnp.float32)

    def zbody(i, carry):
        for kk in range(N_EXP // 16):
            imp2d[i, pl.ds(kk * 16, 16)] = zeros16
            load2d[i, pl.ds(kk * 16, 16)] = zeros16
        return carry

    lax.fori_loop(0, 16, zbody, 0)

    bufs = (buf0, buf1)

    def start_in(ci, buf):
        return pltpu.async_copy(
            lt_hbm.at[:, pl.ds(base + ci * CHUNK, CHUNK)], buf, dsem)

    def process_chunk(ci, buf):
        def gbody(g, carry):
            col0 = g * 16

            bests, bidxs = [], []
            for _r in range(K_SEL):
                def jbody(jj, st):
                    best, bidx = st
                    vals = []
                    for u in range(UNROLL):
                        j = jj * UNROLL + u
                        vals.append((buf[j, pl.ds(col0, 16)],
                                     jnp.full((16,), j, jnp.int32)))
                    while len(vals) > 1:
                        nxt = []
                        for p in range(0, len(vals), 2):
                            nxt.append(_merge(*vals[p], *vals[p + 1]))
                        vals = nxt
                    return _merge(best, bidx, *vals[0])

                best, bidx = lax.fori_loop(
                    0, N_EXP // UNROLL, jbody,
                    (neg_inf, jnp.zeros((16,), jnp.int32)))
                plsc.store_scatter(buf, [bidx, col0 + lanes], neg_inf)
                bests.append(best)
                bidxs.append(bidx)

            v0 = bests[0]
            es = [jnp.exp(b - v0) for b in bests]
            z = es[0]
            for e in es[1:]:
                z = z + e
            rowl = col0 + lanes
            for r in range(K_SEL):
                score = es[r] / z
                rvec = jnp.full((16,), r, jnp.int32)
                plsc.store_scatter(idxb, [rowl, rvec], bidxs[r])
                plsc.store_scatter(scrb, [rowl, rvec], score)
                plsc.addupdate_scatter(imp2d, [lanes, bidxs[r]], score)
                plsc.addupdate_scatter(
                    load2d, [lanes, bidxs[r]],
                    jnp.where(score > 0, jnp.float32(1), jnp.float32(0)))
            return carry

        lax.fori_loop(0, NGRP, gbody, 0)
        row0 = base + ci * CHUNK
        pltpu.sync_copy(idxb, idx_hbm.at[pl.ds(row0, CHUNK), :])
        pltpu.sync_copy(scrb, scr_hbm.at[pl.ds(row0, CHUNK), :])

    pending = start_in(0, bufs[0])
    for ci in range(NCHUNK):
        pending.wait()
        if ci + 1 < NCHUNK:
            pending = start_in(ci + 1, bufs[(ci + 1) % 2])
        process_chunk(ci, bufs[ci % 2])

    pltpu.sync_copy(imp2d, imp_hbm.at[pl.ds(wid * 16, 16), :])
    pltpu.sync_copy(load2d, load_hbm.at[pl.ds(wid * 16, 16), :])


def _route(lt):
    f = pl.kernel(
        _route_body,
        out_type=(
            jax.ShapeDtypeStruct((N_TOK, K_SEL), jnp.int32),
            jax.ShapeDtypeStruct((N_TOK, K_SEL), jnp.float32),
            jax.ShapeDtypeStruct((NW * 16, N_EXP), jnp.float32),
            jax.ShapeDtypeStruct((NW * 16, N_EXP), jnp.float32),
        ),
        mesh=plsc.VectorSubcoreMesh(core_axis_name="c", subcore_axis_name="s"),
        compiler_params=pltpu.CompilerParams(needs_layout_passes=False),
        scratch_types=[
            pltpu.VMEM((N_EXP, CHUNK), jnp.float32),
            pltpu.VMEM((N_EXP, CHUNK), jnp.float32),
            pltpu.VMEM((CHUNK, K_SEL), jnp.int32),
            pltpu.VMEM((CHUNK, K_SEL), jnp.float32),
            pltpu.VMEM((16, N_EXP), jnp.float32),
            pltpu.VMEM((16, N_EXP), jnp.float32),
            pltpu.SemaphoreType.DMA,
        ],
    )
    return f(lt)


def _combine_body(imp_ref, load_ref, loss_ref, load_out, imp_out):
    imp = jnp.sum(imp_ref[...], axis=0, keepdims=True)
    loadf = jnp.sum(load_ref[...], axis=0, keepdims=True)
    imp_out[...] = imp
    load_out[...] = loadf.astype(jnp.int32)

    def cv2(v):
        mean = jnp.mean(v)
        var = jnp.sum((v - mean) ** 2) / (v.size - 1)
        return var / (mean * mean + 1e-10)

    loss_ref[...] = jnp.full((1, 1), 0.01) * (cv2(imp) + cv2(loadf))


def _combine(imp_part, load_part):
    return pl.pallas_call(
        _combine_body,
        out_shape=(
            jax.ShapeDtypeStruct((1, 1), jnp.float32),
            jax.ShapeDtypeStruct((1, N_EXP), jnp.int32),
            jax.ShapeDtypeStruct((1, N_EXP), jnp.float32),
        ),
    )(imp_part, load_part)


@jax.jit
def kernel(x, W1, W2):
    lt = _gate(x, W1, W2)
    idx, scr, imp_part, load_part = _route(lt)
    loss, load, imp = _combine(imp_part, load_part)
    return (idx, scr, loss.reshape(()), load.reshape(N_EXP),
            imp.reshape(N_EXP))
